# baseline (speedup 1.0000x reference)
import functools

import jax
import jax.numpy as jnp
from jax import lax
from jax.experimental import pallas as pl
from jax.experimental.pallas import tpu as pltpu
from jax.experimental.pallas import tpu_sc as plsc

NUM_CORES = 2
NUM_SUBCORES = 16
NUM_WORKERS = NUM_CORES * NUM_SUBCORES


@functools.partial(jax.jit, static_argnums=(2, 3, 4))
def _sc_embed(ids_t, weight, N, S, D):
    n_per_w = N // NUM_WORKERS
    mesh = plsc.VectorSubcoreMesh(
        core_axis_name="c", subcore_axis_name="s",
        num_cores=NUM_CORES, num_subcores=NUM_SUBCORES)

    @functools.partial(
        pl.kernel,
        out_type=jax.ShapeDtypeStruct((S, D, N, 1), jnp.float32),
        mesh=mesh,
        scratch_types=[
            pltpu.VMEM((n_per_w,), jnp.int32),
            pltpu.VMEM((n_per_w,), jnp.int32),
            pltpu.VMEM((n_per_w, D), jnp.float32),
            pltpu.VMEM((n_per_w, D), jnp.float32),
            pltpu.SemaphoreType.DMA,
            pltpu.SemaphoreType.DMA,
            pltpu.SemaphoreType.DMA,
            pltpu.SemaphoreType.DMA,
            pltpu.SemaphoreType.DMA,
            pltpu.SemaphoreType.DMA,
        ],
        compiler_params=pltpu.CompilerParams(use_tc_tiling_on_sc=False),
    )
    def k(ids_hbm, table_hbm, out_hbm, idx0, idx1, rows0, rows1,
          ls0, ls1, gs0, gs1, ws0, ws1):
        wid = lax.axis_index("s") * NUM_CORES + lax.axis_index("c")
        n0 = wid * n_per_w
        L = n_per_w
        idx = [idx0, idx1]
        rows = [rows0, rows1]
        lsem = [ls0, ls1]
        gsem = [gs0, gs1]
        wsem = [ws0, ws1]

        def load(s, b):
            return pltpu.make_async_copy(
                ids_hbm.at[s, pl.ds(n0, L)], idx[b], lsem[b])

        def gath(b):
            return pltpu.make_async_copy(table_hbm.at[idx[b]], rows[b], gsem[b])

        def wbs(s, b):
            return [pltpu.make_async_copy(
                        rows[b].at[:, pl.ds(c, 1)],
                        out_hbm.at[s, c, pl.ds(n0, L), :], wsem[b])
                    for c in range(D)]

        def stage(g, first):
            gath(0).wait()
            if not first:
                for cp in wbs(g - 1, 1):
                    cp.wait()
            load(g + 1, 1).wait()
            gath(1).start()
            load(g + 2, 0).start()
            w0 = wbs(g, 0)
            for cp in w0:
                cp.start()
            gath(1).wait()
            for cp in w0:
                cp.wait()
            load(g + 2, 0).wait()
            gath(0).start()
            load(g + 3, 1).start()
            for cp in wbs(g + 1, 1):
                cp.start()

        load(0, 0).start()
        load(1, 1).start()
        load(0, 0).wait()
        gath(0).start()
        stage(0, first=True)
        lax.fori_loop(
            1, S // 2 - 1, lambda m, c: (stage(2 * m, False), c)[1], 0)
        g = S - 2
        gath(0).wait()
        for cp in wbs(g - 1, 1):
            cp.wait()
        load(g + 1, 1).wait()
        gath(1).start()
        w0 = wbs(g, 0)
        for cp in w0:
            cp.start()
        gath(1).wait()
        for cp in w0:
            cp.wait()
        w1 = wbs(g + 1, 1)
        for cp in w1:
            cp.start()
        for cp in w1:
            cp.wait()

    return k(ids_t, weight)


def kernel(token_ids, weight):
    N, S = token_ids.shape
    V, D = weight.shape
    ids_t = token_ids.T.astype(jnp.int32)
    out_t = _sc_embed(ids_t, weight, N, S, D)
    return jnp.transpose(out_t[..., 0], (2, 0, 1))


# speedup vs baseline: 34.8390x; 34.8390x over previous
import functools

import jax
import jax.numpy as jnp
from jax import lax
from jax.experimental import pallas as pl
from jax.experimental.pallas import tpu as pltpu
from jax.experimental.pallas import tpu_sc as plsc

NUM_CORES = 2
NUM_SUBCORES = 16
NUM_WORKERS = NUM_CORES * NUM_SUBCORES
CHUNK = 1280


@functools.partial(jax.jit, static_argnums=(2, 3))
def _sc_embed(flat_ids, weight, B, D):
    b_per_w = B // NUM_WORKERS
    n_chunks = b_per_w // CHUNK
    mesh = plsc.VectorSubcoreMesh(
        core_axis_name="c", subcore_axis_name="s",
        num_cores=NUM_CORES, num_subcores=NUM_SUBCORES)

    @functools.partial(
        pl.kernel,
        out_type=jax.ShapeDtypeStruct((B, D), jnp.float32),
        mesh=mesh,
        scratch_types=[
            pltpu.VMEM((CHUNK,), jnp.int32),
            pltpu.VMEM((CHUNK,), jnp.int32),
            pltpu.VMEM((CHUNK, D), jnp.float32),
            pltpu.VMEM((CHUNK, D), jnp.float32),
            pltpu.SemaphoreType.DMA,
            pltpu.SemaphoreType.DMA,
            pltpu.SemaphoreType.DMA,
            pltpu.SemaphoreType.DMA,
            pltpu.SemaphoreType.DMA,
            pltpu.SemaphoreType.DMA,
        ],
        compiler_params=pltpu.CompilerParams(use_tc_tiling_on_sc=False),
    )
    def k(idx_hbm, table_hbm, out_hbm, idx0, idx1, rows0, rows1,
          ls0, ls1, gs0, gs1, ws0, ws1):
        wid = lax.axis_index("s") * NUM_CORES + lax.axis_index("c")
        base = wid * b_per_w
        idx = [idx0, idx1]
        rows = [rows0, rows1]
        lsem = [ls0, ls1]
        gsem = [gs0, gs1]
        wsem = [ws0, ws1]

        def load(i):
            b = i % 2
            return pltpu.async_copy(
                idx_hbm.at[pl.ds(base + i * CHUNK, CHUNK)], idx[b], lsem[b])

        def gather(i):
            b = i % 2
            return pltpu.async_copy(table_hbm.at[idx[b]], rows[b], gsem[b])

        def writeback(i):
            b = i % 2
            return pltpu.async_copy(
                rows[b], out_hbm.at[pl.ds(base + i * CHUNK, CHUNK)], wsem[b])

        loads = [None] * n_chunks
        gathers = [None] * n_chunks
        writes = [None] * n_chunks
        loads[0] = load(0)
        if n_chunks > 1:
            loads[1] = load(1)
        for i in range(n_chunks):
            loads[i].wait()
            if i >= 2:
                writes[i - 2].wait()
            gathers[i] = gather(i)
            if i >= 1:
                gathers[i - 1].wait()
                writes[i - 1] = writeback(i - 1)
                if i + 1 < n_chunks:
                    loads[i + 1] = load(i + 1)
        gathers[n_chunks - 1].wait()
        writes[n_chunks - 1] = writeback(n_chunks - 1)
        if n_chunks > 1:
            writes[n_chunks - 2].wait()
        writes[n_chunks - 1].wait()

    return k(flat_ids, weight)


def kernel(token_ids, weight):
    N, S = token_ids.shape
    B = N * S
    D = weight.shape[1]
    flat = token_ids.reshape(B).astype(jnp.int32)
    out = _sc_embed(flat, weight, B, D)
    return out.reshape(N, S, D)
